# baseline (speedup 1.0000x reference)
import jax
import jax.numpy as jnp
from jax.experimental import pallas as pl
from jax.experimental.pallas import tpu as pltpu

_DIM = 16
_MOM = 0.999
_TEMP = 0.07
_EPS = 1e-8
_SPATIAL = 16 * 16 * 16

_NWIN = 8
_B = 4
_NSTEPS = _NWIN * _B


def _fused_kernel(trg_ref, imq_ref, imk_ref, wq_ref, bq_ref, wk_ref, bk_ref,
                  qt_ref, pos_ref, ln_ref, qtout_ref, kvt_ref,
                  acc_ref, an_ref):
    i = pl.program_id(0)
    b = jax.lax.rem(i, _B)

    @pl.when(i == 0)
    def _init():
        af = jnp.mean(trg_ref[...], axis=(2, 3, 4))
        anchor = af @ wq_ref[...] + bq_ref[...][None, :]
        a_n = anchor / jnp.maximum(
            jnp.sqrt(jnp.sum(anchor * anchor, axis=1, keepdims=True)), _EPS)
        an_ref[...] = a_n * (1.0 / _TEMP)
        qf = jnp.mean(imq_ref[...], axis=(2, 3, 4))
        qv = qf @ wk_ref[...] + bk_ref[...][None, :]
        q_n = qv / jnp.maximum(
            jnp.sqrt(jnp.sum(qv * qv, axis=1, keepdims=True)), _EPS)
        pos_ref[...] = jnp.zeros_like(pos_ref)
        pos_ref[0:_B, 0:1] = jnp.sum(an_ref[...] * q_n, axis=1, keepdims=True)
        acc_ref[...] = jnp.zeros_like(acc_ref)

    @pl.when(i < 8)
    def _imk():
        sums = jnp.sum(imk_ref[...], axis=2)
        prow = jax.lax.broadcasted_iota(jnp.int32, (8, 64), 0)
        pcol = jax.lax.broadcasted_iota(jnp.int32, (8, 64), 1)
        place = (pcol == i * 8 + prow).astype(jnp.float32)
        acc_ref[...] += jnp.dot(sums, place,
                                preferred_element_type=jnp.float32)

    x = qt_ref[...]
    a_row = an_ref[pl.ds(b, 1), :]
    dots = jnp.dot(a_row, x, preferred_element_type=jnp.float32)
    sumsq = jnp.dot(jnp.full((1, _DIM), 1.0, jnp.float32), x * x,
                    preferred_element_type=jnp.float32)
    ln_ref[pl.ds(b, 1), :] = dots * jax.lax.rsqrt(
        jnp.maximum(sumsq, _EPS * _EPS))
    qtout_ref[...] = x

    @pl.when(i == _NSTEPS - 1)
    def _enqueue():
        kft = acc_ref[...] * (1.0 / _SPATIAL)
        wk2 = wk_ref[...] * _MOM + wq_ref[...] * (1.0 - _MOM)
        bk2 = bk_ref[...] * _MOM + bq_ref[...] * (1.0 - _MOM)
        kvt_ref[...] = jax.lax.dot_general(
            wk2, kft, (((0,), (0,)), ((), ())),
            preferred_element_type=jnp.float32) + bk2[:, None]


def kernel(trg_anchor, im_q, im_k, Wq, bq, Wk, bk, src_queue):
    nrows = src_queue.shape[0]
    ch = nrows // (_NWIN * _B)
    nk = im_k.shape[0] * im_k.shape[1]

    imk = jnp.transpose(im_k.reshape(nk, _DIM, _SPATIAL), (1, 0, 2))
    qt = src_queue.T

    f32 = jnp.float32

    def _qt_map(i):
        return (0, jax.lax.rem(i, _B) * _NWIN + jax.lax.div(i, _B))

    pos, ln, qtout, kvt = pl.pallas_call(
        _fused_kernel,
        grid=(_NSTEPS,),
        in_specs=[
            pl.BlockSpec((_B, _DIM, 16, 16, 16), lambda i: (0, 0, 0, 0, 0)),
            pl.BlockSpec((_B, _DIM, 16, 16, 16), lambda i: (0, 0, 0, 0, 0)),
            pl.BlockSpec((_DIM, 8, _SPATIAL),
                         lambda i: (0, jnp.minimum(i, 7), 0)),
            pl.BlockSpec((_DIM, _DIM), lambda i: (0, 0)),
            pl.BlockSpec((_DIM,), lambda i: (0,)),
            pl.BlockSpec((_DIM, _DIM), lambda i: (0, 0)),
            pl.BlockSpec((_DIM,), lambda i: (0,)),
            pl.BlockSpec((_DIM, ch), _qt_map),
        ],
        out_specs=[
            pl.BlockSpec((8, 128), lambda i: (0, 0)),
            pl.BlockSpec((8, ch), lambda i: (0, jax.lax.div(i, _B))),
            pl.BlockSpec((_DIM, ch), _qt_map),
            pl.BlockSpec((_DIM, 64), lambda i: (0, 0)),
        ],
        out_shape=[
            jax.ShapeDtypeStruct((8, 128), f32),
            jax.ShapeDtypeStruct((8, _NWIN * ch), f32),
            jax.ShapeDtypeStruct((_DIM, nrows), f32),
            jax.ShapeDtypeStruct((_DIM, 64), f32),
        ],
        scratch_shapes=[
            pltpu.VMEM((_DIM, nk), f32),
            pltpu.VMEM((_B, _DIM), f32),
        ],
    )(trg_anchor, im_q, imk, Wq, bq, Wk, bk, qt)

    nqt = jax.lax.dynamic_update_slice(qtout, kvt, (0, 0))
    nq = jnp.transpose(nqt)
    logits = jnp.concatenate([pos[:_B, :1], ln[:_B, :]], axis=1)
    labels = jnp.zeros((_B,), jnp.int32)
    return (logits, labels, nq)


# speedup vs baseline: 2.6834x; 2.6834x over previous
import jax
import jax.numpy as jnp
from jax.experimental import pallas as pl
from jax.experimental.pallas import tpu as pltpu

_DIM = 16
_MOM = 0.999
_TEMP = 0.07
_EPS = 1e-8
_SPATIAL = 16 * 16 * 16

_NWIN = 8
_B = 4
_NSTEPS = _NWIN * _B


def _fused_kernel(trg_ref, imq_ref, imk_ref, wq_ref, bq_ref, wk_ref, bk_ref,
                  qt_ref, pos_ref, ln_ref, qtout_ref, kvt_ref,
                  acc_ref, an_ref):
    i = pl.program_id(0)
    b = jax.lax.rem(i, _B)

    @pl.when(i == 0)
    def _init():
        af = jnp.mean(trg_ref[...], axis=(2, 3, 4))
        anchor = af @ wq_ref[...] + bq_ref[...][None, :]
        a_n = anchor / jnp.maximum(
            jnp.sqrt(jnp.sum(anchor * anchor, axis=1, keepdims=True)), _EPS)
        an_ref[...] = a_n * (1.0 / _TEMP)
        qf = jnp.mean(imq_ref[...], axis=(2, 3, 4))
        qv = qf @ wk_ref[...] + bk_ref[...][None, :]
        q_n = qv / jnp.maximum(
            jnp.sqrt(jnp.sum(qv * qv, axis=1, keepdims=True)), _EPS)
        pos_ref[...] = jnp.zeros_like(pos_ref)
        pos_ref[0:_B, 0:1] = jnp.sum(an_ref[...] * q_n, axis=1, keepdims=True)

    rps = 64 // _NSTEPS
    acc_ref[pl.ds(i * rps, rps), :] = jnp.sum(imk_ref[...], axis=(2, 3, 4))

    x = qt_ref[...]
    a_row = an_ref[pl.ds(b, 1), :]
    dots = jnp.dot(a_row, x, preferred_element_type=jnp.float32)
    sumsq = jnp.dot(jnp.full((1, _DIM), 1.0, jnp.float32), x * x,
                    preferred_element_type=jnp.float32)
    ln_ref[pl.ds(b, 1), :] = dots * jax.lax.rsqrt(
        jnp.maximum(sumsq, _EPS * _EPS))
    qtout_ref[...] = x

    @pl.when(i == _NSTEPS - 1)
    def _enqueue():
        kf = acc_ref[...] * (1.0 / _SPATIAL)
        wk2 = wk_ref[...] * _MOM + wq_ref[...] * (1.0 - _MOM)
        bk2 = bk_ref[...] * _MOM + bq_ref[...] * (1.0 - _MOM)
        kv = kf @ wk2 + bk2[None, :]
        eye = (jax.lax.broadcasted_iota(jnp.int32, (_DIM, _DIM), 0) ==
               jax.lax.broadcasted_iota(jnp.int32, (_DIM, _DIM), 1)
               ).astype(jnp.float32)
        kvt_ref[...] = jax.lax.dot_general(
            eye, kv, (((1,), (1,)), ((), ())),
            preferred_element_type=jnp.float32)


def kernel(trg_anchor, im_q, im_k, Wq, bq, Wk, bk, src_queue):
    nrows = src_queue.shape[0]
    ch = nrows // (_NWIN * _B)
    nk = im_k.shape[0] * im_k.shape[1]

    imk = im_k.reshape(nk, _DIM, 16, 16, 16)
    qt = src_queue.T

    f32 = jnp.float32

    def _qt_map(i):
        return (0, jax.lax.rem(i, _B) * _NWIN + jax.lax.div(i, _B))

    pos, ln, qtout, kvt = pl.pallas_call(
        _fused_kernel,
        grid=(_NSTEPS,),
        in_specs=[
            pl.BlockSpec((_B, _DIM, 16, 16, 16), lambda i: (0, 0, 0, 0, 0)),
            pl.BlockSpec((_B, _DIM, 16, 16, 16), lambda i: (0, 0, 0, 0, 0)),
            pl.BlockSpec((nk // _NSTEPS, _DIM, 16, 16, 16),
                         lambda i: (i, 0, 0, 0, 0)),
            pl.BlockSpec((_DIM, _DIM), lambda i: (0, 0)),
            pl.BlockSpec((_DIM,), lambda i: (0,)),
            pl.BlockSpec((_DIM, _DIM), lambda i: (0, 0)),
            pl.BlockSpec((_DIM,), lambda i: (0,)),
            pl.BlockSpec((_DIM, ch), _qt_map),
        ],
        out_specs=[
            pl.BlockSpec((8, 128), lambda i: (0, 0)),
            pl.BlockSpec((8, ch), lambda i: (0, jax.lax.div(i, _B))),
            pl.BlockSpec((_DIM, ch), _qt_map),
            pl.BlockSpec((_DIM, 64), lambda i: (0, 0)),
        ],
        out_shape=[
            jax.ShapeDtypeStruct((8, 128), f32),
            jax.ShapeDtypeStruct((8, _NWIN * ch), f32),
            jax.ShapeDtypeStruct((_DIM, nrows), f32),
            jax.ShapeDtypeStruct((_DIM, 64), f32),
        ],
        scratch_shapes=[
            pltpu.VMEM((nk, _DIM), f32),
            pltpu.VMEM((_B, _DIM), f32),
        ],
    )(trg_anchor, im_q, imk, Wq, bq, Wk, bk, qt)

    nqt = jax.lax.dynamic_update_slice(qtout, kvt, (0, 0))
    nq = jnp.transpose(nqt)
    logits = jnp.concatenate([pos[:_B, :1], ln[:_B, :]], axis=1)
    labels = jnp.zeros((_B,), jnp.int32)
    return (logits, labels, nq)
